# baseline (speedup 1.0000x reference)
import functools

import numpy as np
import jax
import jax.numpy as jnp
from jax import lax
from jax.experimental import pallas as pl
from jax.experimental.pallas import tpu as pltpu
from jax.experimental.pallas import tpu_sc as plsc

SEQ_LEN = 2048
VOCAB = 100000
D_MODEL = 1024
BATCH = 4

NW = 32
ROWS = BATCH * SEQ_LEN
ROWS_PER_W = ROWS // NW
CHUNK = 32
NCHUNK = ROWS_PER_W // CHUNK
LANES = 16
VPR = D_MODEL // LANES


def _pos_encoding() -> np.ndarray:
    pos = np.arange(SEQ_LEN)[:, None].astype(np.float32)
    i = np.arange(D_MODEL)[None, :]
    angle_rates = 1.0 / np.power(10000.0, (2.0 * (i // 2)) / np.float32(D_MODEL))
    angles = pos * angle_rates
    return np.where(i % 2 == 0, np.sin(angles), np.cos(angles)).astype(np.float32)


_PE = _pos_encoding()


_MESH = plsc.VectorSubcoreMesh(core_axis_name="c", subcore_axis_name="s")


@functools.partial(
    pl.kernel,
    mesh=_MESH,
    out_type=[
        jax.ShapeDtypeStruct((ROWS, D_MODEL), jnp.float32),
        jax.ShapeDtypeStruct((ROWS, D_MODEL), jnp.float32),
    ],
    scratch_types=[
        pltpu.VMEM((CHUNK,), jnp.int32),
        pltpu.VMEM((CHUNK,), jnp.int32),
        pltpu.VMEM((CHUNK, D_MODEL), jnp.float32),
        pltpu.VMEM((CHUNK, D_MODEL), jnp.float32),
        pltpu.VMEM((CHUNK, D_MODEL), jnp.float32),
        pltpu.SemaphoreType.DMA,
        pltpu.SemaphoreType.DMA,
    ],
)
def _emb_kernel(x_hbm, xo_hbm, pe_hbm, tab_hbm, enc_hbm, dec_hbm,
                idx_e, idx_d, pe_v, emb_e, emb_d, sem_e, sem_d):
    wid = lax.axis_index("s") * 2 + lax.axis_index("c")
    base = wid * ROWS_PER_W
    s_base = base % SEQ_LEN

    def chunk_body(c, carry):
        rb = base + c * CHUNK
        sb = s_base + c * CHUNK
        pltpu.sync_copy(x_hbm.at[pl.ds(rb, CHUNK)], idx_e)
        pltpu.sync_copy(xo_hbm.at[pl.ds(rb, CHUNK)], idx_d)
        ce = pltpu.async_copy(tab_hbm.at[idx_e], emb_e, sem_e)
        cd = pltpu.async_copy(tab_hbm.at[idx_d], emb_d, sem_d)
        pltpu.sync_copy(pe_hbm.at[pl.ds(sb, CHUNK)], pe_v)
        ce.wait()
        cd.wait()

        def row_body(r, rcarry):
            for j in range(VPR):
                sl = pl.ds(j * LANES, LANES)
                pv = pe_v[r, sl]
                emb_e[r, sl] += pv
                emb_d[r, sl] += pv
            return rcarry

        lax.fori_loop(0, CHUNK, row_body, 0)
        pltpu.sync_copy(emb_e, enc_hbm.at[pl.ds(rb, CHUNK)])
        pltpu.sync_copy(emb_d, dec_hbm.at[pl.ds(rb, CHUNK)])
        return carry

    lax.fori_loop(0, NCHUNK, chunk_body, 0)


def kernel(x, x_output, emb_table):
    enc, dec = _emb_kernel(
        x.reshape(ROWS), x_output.reshape(ROWS), jnp.asarray(_PE), emb_table
    )
    return (
        enc.reshape(BATCH, SEQ_LEN, D_MODEL),
        dec.reshape(BATCH, SEQ_LEN, D_MODEL),
    )
